# baseline (speedup 1.0000x reference)
import functools

import jax
import jax.numpy as jnp
from jax import lax
from jax.experimental import pallas as pl
from jax.experimental.pallas import tpu as pltpu
from jax.experimental.pallas import tpu_sc as plsc

H = 1080
W = 1920
HW = H * W
N = 2_000_000

NW = 32
G_MAIN = 64_000
G_TAIL = N - 31 * G_MAIN
CHUNK = 8_000
NCH_MAIN = G_MAIN // CHUNK
NCH_TAIL = G_TAIL // CHUNK

ROWS = N // 128
BLK = 512
GRID = -(-ROWS // BLK)


def _sc_gather_body(p0, p1, p2, c_hbm, m_hbm, pix_hbm, g0, g1, g2,
                    cv, mv, pv, gv, b0, b1, b2, s0, s1, s2):
    wid = lax.axis_index("s") * 2 + lax.axis_index("c")
    base = wid * G_MAIN

    def mask_idx(it, _):
        sl = pl.ds(it * 16, 16)
        win = cv[sl] > mv[sl]
        gv[sl] = jnp.where(win, pv[sl], 0)
        return 0

    for j in range(NCH_MAIN):
        @pl.when(jnp.logical_or(wid < 31, j < NCH_TAIL))
        def _():
            off = base + j * CHUNK
            pltpu.sync_copy(pix_hbm.at[pl.ds(off, CHUNK)], pv)
            pltpu.sync_copy(c_hbm.at[pl.ds(off, CHUNK)], cv)
            pltpu.sync_copy(m_hbm.at[pl.ds(off, CHUNK)], mv)
            lax.fori_loop(0, CHUNK // 16, mask_idx, 0)
            cp0 = pltpu.async_copy(p0.at[gv], b0, s0)
            cp1 = pltpu.async_copy(p1.at[gv], b1, s1)
            cp2 = pltpu.async_copy(p2.at[gv], b2, s2)
            cp0.wait()
            cp1.wait()
            cp2.wait()
            pltpu.sync_copy(b0, g0.at[pl.ds(off, CHUNK)])
            pltpu.sync_copy(b1, g1.at[pl.ds(off, CHUNK)])
            pltpu.sync_copy(b2, g2.at[pl.ds(off, CHUNK)])


_sc_gather = functools.partial(
    pl.kernel,
    mesh=plsc.VectorSubcoreMesh(core_axis_name="c", subcore_axis_name="s"),
    out_type=[jax.ShapeDtypeStruct((N,), jnp.float32)] * 3,
    scratch_types=[
        pltpu.VMEM((CHUNK,), jnp.float32),
        pltpu.VMEM((CHUNK,), jnp.float32),
        pltpu.VMEM((CHUNK,), jnp.int32),
        pltpu.VMEM((CHUNK,), jnp.int32),
        pltpu.VMEM((CHUNK,), jnp.float32),
        pltpu.VMEM((CHUNK,), jnp.float32),
        pltpu.VMEM((CHUNK,), jnp.float32),
        pltpu.SemaphoreType.DMA,
        pltpu.SemaphoreType.DMA,
        pltpu.SemaphoreType.DMA,
    ],
)(_sc_gather_body)


def _ew_body(c_ref, s_ref, m_ref, t_ref, dmin_ref, g0_ref, g1_ref, g2_ref,
             oldt_ref, nmax_ref, ntot_ref, nmin_ref, ncolt_ref):
    c = c_ref[...]
    m = m_ref[...]
    mask = c > m
    nmax_ref[...] = jnp.where(mask, c, m)
    ntot_ref[...] = t_ref[...] + c
    s = s_ref[...]
    d = dmin_ref[...]
    nmin_ref[...] = jnp.where(s < d, s, d)
    ncolt_ref[0] = jnp.where(mask, g0_ref[...], oldt_ref[0])
    ncolt_ref[1] = jnp.where(mask, g1_ref[...], oldt_ref[1])
    ncolt_ref[2] = jnp.where(mask, g2_ref[...], oldt_ref[2])


def _ew_call(c, s, m, t, dmin, g0, g1, g2, oldt):
    flat_spec = pl.BlockSpec((BLK, 128), lambda i: (i, 0))
    col_spec = pl.BlockSpec((3, BLK, 128), lambda i: (0, i, 0))
    return pl.pallas_call(
        _ew_body,
        grid=(GRID,),
        in_specs=[flat_spec] * 8 + [col_spec],
        out_specs=[flat_spec] * 3 + [col_spec],
        out_shape=[
            jax.ShapeDtypeStruct((ROWS, 128), jnp.float32),
            jax.ShapeDtypeStruct((ROWS, 128), jnp.float32),
            jax.ShapeDtypeStruct((ROWS, 128), jnp.float32),
            jax.ShapeDtypeStruct((3, ROWS, 128), jnp.float32),
        ],
    )(c, s, m, t, dmin, g0, g1, g2, oldt)


def kernel(colour, current_gauss_contributions, current_gauss_surface_distances,
           gaussian_max_contribution, gaussian_colours, gaussian_total_contribution,
           gaussian_min_surface_distance, current_gauss_pixels):
    planes = colour.reshape(3, HW)
    g0, g1, g2 = _sc_gather(planes[0], planes[1], planes[2],
                            current_gauss_contributions,
                            gaussian_max_contribution,
                            current_gauss_pixels)

    r = lambda x: x.reshape(ROWS, 128)
    oldt = gaussian_colours.T.reshape(3, ROWS, 128)
    nmax, ntot, nmin, ncolt = _ew_call(
        r(current_gauss_contributions),
        r(current_gauss_surface_distances),
        r(gaussian_max_contribution),
        r(gaussian_total_contribution),
        r(gaussian_min_surface_distance),
        r(g0), r(g1), r(g2), oldt)

    new_colours = ncolt.reshape(3, N).T
    return (nmax.reshape(N), new_colours, ntot.reshape(N), nmin.reshape(N))


# speedup vs baseline: 1.0062x; 1.0062x over previous
import functools

import jax
import jax.numpy as jnp
from jax import lax
from jax.experimental import pallas as pl
from jax.experimental.pallas import tpu as pltpu
from jax.experimental.pallas import tpu_sc as plsc

H = 1080
W = 1920
HW = H * W
N = 2_000_000

NW = 32
G_MAIN = 64_000
G_TAIL = N - 31 * G_MAIN
CHUNK = 8_000
NCH_MAIN = G_MAIN // CHUNK
NCH_TAIL = G_TAIL // CHUNK

ROWS = N // 128
BLK = 512
GRID = -(-ROWS // BLK)


def _sc_gather_body(p0, p1, p2, c_hbm, m_hbm, pix_hbm, g0, g1, g2,
                    cv, mv, pv, gv, b0, b1, b2, s0, s1, s2):
    wid = lax.axis_index("s") * 2 + lax.axis_index("c")
    base = wid * G_MAIN

    def mask_idx(it, _):
        sl = pl.ds(it * 16, 16)
        win = cv[sl] > mv[sl]
        gv[sl] = jnp.where(win, pv[sl], 0)
        return 0

    for j in range(NCH_MAIN):
        @pl.when(jnp.logical_or(wid < 31, j < NCH_TAIL))
        def _():
            off = base + j * CHUNK
            pltpu.sync_copy(pix_hbm.at[pl.ds(off, CHUNK)], pv)
            pltpu.sync_copy(c_hbm.at[pl.ds(off, CHUNK)], cv)
            pltpu.sync_copy(m_hbm.at[pl.ds(off, CHUNK)], mv)
            lax.fori_loop(0, CHUNK // 16, mask_idx, 0)
            cp0 = pltpu.async_copy(p0.at[gv], b0, s0)
            cp1 = pltpu.async_copy(p1.at[gv], b1, s1)
            cp2 = pltpu.async_copy(p2.at[gv], b2, s2)
            cp0.wait()
            cp1.wait()
            cp2.wait()
            pltpu.sync_copy(b0, g0.at[pl.ds(off, CHUNK)])
            pltpu.sync_copy(b1, g1.at[pl.ds(off, CHUNK)])
            pltpu.sync_copy(b2, g2.at[pl.ds(off, CHUNK)])


_sc_gather = functools.partial(
    pl.kernel,
    mesh=plsc.VectorSubcoreMesh(core_axis_name="c", subcore_axis_name="s"),
    compiler_params=pltpu.CompilerParams(needs_layout_passes=False),
    out_type=[jax.ShapeDtypeStruct((N,), jnp.float32)] * 3,
    scratch_types=[
        pltpu.VMEM((CHUNK,), jnp.float32),
        pltpu.VMEM((CHUNK,), jnp.float32),
        pltpu.VMEM((CHUNK,), jnp.int32),
        pltpu.VMEM((CHUNK,), jnp.int32),
        pltpu.VMEM((CHUNK,), jnp.float32),
        pltpu.VMEM((CHUNK,), jnp.float32),
        pltpu.VMEM((CHUNK,), jnp.float32),
        pltpu.SemaphoreType.DMA,
        pltpu.SemaphoreType.DMA,
        pltpu.SemaphoreType.DMA,
    ],
)(_sc_gather_body)


def _ew_body(c_ref, s_ref, m_ref, t_ref, dmin_ref, g0_ref, g1_ref, g2_ref,
             oldt_ref, nmax_ref, ntot_ref, nmin_ref, ncolt_ref):
    c = c_ref[...]
    m = m_ref[...]
    mask = c > m
    nmax_ref[...] = jnp.where(mask, c, m)
    ntot_ref[...] = t_ref[...] + c
    s = s_ref[...]
    d = dmin_ref[...]
    nmin_ref[...] = jnp.where(s < d, s, d)
    ncolt_ref[0] = jnp.where(mask, g0_ref[...], oldt_ref[0])
    ncolt_ref[1] = jnp.where(mask, g1_ref[...], oldt_ref[1])
    ncolt_ref[2] = jnp.where(mask, g2_ref[...], oldt_ref[2])


def _ew_call(c, s, m, t, dmin, g0, g1, g2, oldt):
    flat_spec = pl.BlockSpec((BLK, 128), lambda i: (i, 0))
    col_spec = pl.BlockSpec((3, BLK, 128), lambda i: (0, i, 0))
    return pl.pallas_call(
        _ew_body,
        grid=(GRID,),
        in_specs=[flat_spec] * 8 + [col_spec],
        out_specs=[flat_spec] * 3 + [col_spec],
        out_shape=[
            jax.ShapeDtypeStruct((ROWS, 128), jnp.float32),
            jax.ShapeDtypeStruct((ROWS, 128), jnp.float32),
            jax.ShapeDtypeStruct((ROWS, 128), jnp.float32),
            jax.ShapeDtypeStruct((3, ROWS, 128), jnp.float32),
        ],
    )(c, s, m, t, dmin, g0, g1, g2, oldt)


def kernel(colour, current_gauss_contributions, current_gauss_surface_distances,
           gaussian_max_contribution, gaussian_colours, gaussian_total_contribution,
           gaussian_min_surface_distance, current_gauss_pixels):
    planes = colour.reshape(3, HW)
    g0, g1, g2 = _sc_gather(planes[0], planes[1], planes[2],
                            current_gauss_contributions,
                            gaussian_max_contribution,
                            current_gauss_pixels)

    r = lambda x: x.reshape(ROWS, 128)
    oldt = gaussian_colours.T.reshape(3, ROWS, 128)
    nmax, ntot, nmin, ncolt = _ew_call(
        r(current_gauss_contributions),
        r(current_gauss_surface_distances),
        r(gaussian_max_contribution),
        r(gaussian_total_contribution),
        r(gaussian_min_surface_distance),
        r(g0), r(g1), r(g2), oldt)

    new_colours = ncolt.reshape(3, N).T
    return (nmax.reshape(N), new_colours, ntot.reshape(N), nmin.reshape(N))
